# baseline (speedup 1.0000x reference)
import functools

import jax
import jax.numpy as jnp
from jax import lax
from jax.experimental import pallas as pl
from jax.experimental.pallas import tpu as pltpu
from jax.experimental.pallas import tpu_sc as plsc

D = 128
B, L = 16384, 50
N = B * L
NC, NS = 2, 16
NW = NC * NS
CH = 128
NG = N // (CH * NW)
PW = 2
NP = NG // PW
NPB = 3


def _emb_body(x_hbm, w_hbm, out_hbm, idx_v, rows_v,
              sg0, sg1, sg2, sw0, sw1, sw2):
    semg = (sg0, sg1, sg2)
    semw = (sw0, sw1, sw2)
    wid = lax.axis_index("s") * NC + lax.axis_index("c")
    gbase = wid * NG

    pltpu.sync_copy(x_hbm.at[pl.ds(gbase, NG)], idx_v)

    def fire_gp(p, b):
        for h in range(PW):
            pltpu.async_copy(
                w_hbm.at[idx_v.at[PW * p + h]], rows_v.at[b, h], semg[b])

    def wait_gp(p, b):
        for h in range(PW):
            pltpu.make_async_copy(
                w_hbm.at[idx_v.at[PW * p + h]], rows_v.at[b, h], semg[b]
            ).wait()

    def fire_w(p, b):
        pltpu.async_copy(
            rows_v.at[b], out_hbm.at[pl.ds(gbase + PW * p, PW)], semw[b])

    def wait_w(p, b):
        pltpu.make_async_copy(
            rows_v.at[b], out_hbm.at[pl.ds(gbase + PW * p, PW)], semw[b]
        ).wait()

    fire_gp(0, 0)
    wait_gp(0, 0); fire_w(0, 0); fire_gp(1, 1)
    wait_gp(1, 1); fire_w(1, 1); fire_gp(2, 2)

    INT = ((NP - 2 - 2) // NPB) * NPB

    @pl.loop(2, 2 + INT, step=NPB)
    def visit_loop(p0):
        for k in range(NPB):
            p = p0 + k
            b = (2 + k) % NPB
            bn = (b + 1) % NPB
            wait_gp(p, b)
            fire_w(p, b)
            wait_w(p - 2, bn)
            fire_gp(p + 1, bn)

    for p in range(2 + INT, NP):
        b = p % NPB
        bn = (b + 1) % NPB
        wait_gp(p, b)
        fire_w(p, b)
        wait_w(p - 2, bn)
        if p + 1 < NP:
            fire_gp(p + 1, bn)
    wait_w(NP - 2, (NP - 2) % NPB)
    wait_w(NP - 1, (NP - 1) % NPB)


@jax.jit
def _emb_lookup(xf, weight):
    mesh = plsc.VectorSubcoreMesh(core_axis_name="c", subcore_axis_name="s")
    run = pl.kernel(
        _emb_body,
        out_type=jax.ShapeDtypeStruct((NW * NG, CH, D), jnp.float32),
        mesh=mesh,
        scratch_types=[
            pltpu.VMEM((NG, CH), jnp.int32),
            pltpu.VMEM((NPB, PW, CH, D), jnp.float32),
        ] + [pltpu.SemaphoreType.DMA] * (2 * NPB),
        name="emb_gather",
    )
    return run(xf, weight)


def kernel(x, weight):
    xf = jnp.transpose(x).reshape(NW * NG, CH).astype(jnp.int32)
    out = _emb_lookup(xf, weight)
    return out.reshape(L, B, D).transpose(1, 0, 2)
